# initial kernel scaffold (unmeasured)
import jax
import jax.numpy as jnp
from jax import lax
from jax.experimental import pallas as pl
from jax.experimental.pallas import tpu as pltpu

N_DEV = 4
E_LOC = 4


def kernel(x, router_W, route_idx, expert_W, shared_W):
    n_tok, d_model = x.shape
    h_dim = shared_W.shape[1]

    def body(x_ref, rw_ref, ridx_ref, ew_ref, sw_ref, out_ref,
             xcomm, pcomm, rcomm, psend, precv,
             send_sems, recv_sems, psend_sems, precv_sems, pg_ref):
        my = lax.axis_index("i")
        left = lax.rem(my + N_DEV - 1, N_DEV)
        right = lax.rem(my + 1, N_DEV)

        bsem = pltpu.get_barrier_semaphore()
        for nbr in (left, right):
            pl.semaphore_signal(
                bsem, inc=1,
                device_id=(nbr,), device_id_type=pl.DeviceIdType.MESH,
            )
        pl.semaphore_wait(bsem, 2)

        xv = x_ref[...]
        scores = jnp.dot(xv, rw_ref[...], preferred_element_type=jnp.float32)
        smax = jnp.max(scores, axis=-1, keepdims=True)
        ex = jnp.exp(scores - smax)
        probs = ex / jnp.sum(ex, axis=-1, keepdims=True)
        iota = lax.broadcasted_iota(jnp.int32, scores.shape, 1)
        ridx = ridx_ref[...]
        pg_ref[...] = jnp.sum(
            jnp.where(iota == ridx, probs, 0.0), axis=-1, keepdims=True
        )

        def expert_partial(xo, po, ro):
            acc = None
            for j in range(E_LOC):
                gate = jnp.where(ro == E_LOC * my + j, po, 0.0)
                contrib = jnp.dot(
                    xo * gate, ew_ref[j], preferred_element_type=jnp.float32
                )
                acc = contrib if acc is None else acc + contrib
            return acc

        def make_hop(h):
            if h == 0:
                srcs = (x_ref, pg_ref, ridx_ref)
            else:
                srcs = (xcomm.at[h - 1], pcomm.at[h - 1], rcomm.at[h - 1])
            dsts = (xcomm.at[h], pcomm.at[h], rcomm.at[h])
            rdmas = []
            for k in range(3):
                rdmas.append(pltpu.make_async_remote_copy(
                    src_ref=srcs[k],
                    dst_ref=dsts[k],
                    send_sem=send_sems.at[h, k],
                    recv_sem=recv_sems.at[h, k],
                    device_id=(right,),
                    device_id_type=pl.DeviceIdType.MESH,
                ))
            return rdmas

        hop = make_hop(0)
        for r in hop:
            r.start()

        out_ref[...] = (
            jnp.dot(xv, sw_ref[...], preferred_element_type=jnp.float32)
            + expert_partial(xv, pg_ref[...], ridx)
        )

        partial_rdmas = []
        for h in range(N_DEV - 1):
            for r in hop:
                r.wait()
            if h < N_DEV - 2:
                hop = make_hop(h + 1)
                for r in hop:
                    r.start()
            origin = lax.rem(my + N_DEV - 1 - h, N_DEV)
            acc = expert_partial(xcomm[h], pcomm[h], rcomm[h])
            psend[h] = acc
            rp = pltpu.make_async_remote_copy(
                src_ref=psend.at[h],
                dst_ref=precv.at[h],
                send_sem=psend_sems.at[h],
                recv_sem=precv_sems.at[h],
                device_id=(origin,),
                device_id_type=pl.DeviceIdType.MESH,
            )
            rp.start()
            partial_rdmas.append(rp)

        for rp in partial_rdmas:
            rp.wait()
        out_ref[...] = out_ref[...] + precv[0] + precv[1] + precv[2]

    return pl.pallas_call(
        body,
        out_shape=jax.ShapeDtypeStruct((n_tok, h_dim), jnp.float32),
        in_specs=[pl.BlockSpec(memory_space=pltpu.VMEM)] * 5,
        out_specs=pl.BlockSpec(memory_space=pltpu.VMEM),
        scratch_shapes=[
            pltpu.VMEM((N_DEV - 1, n_tok, d_model), jnp.float32),
            pltpu.VMEM((N_DEV - 1, n_tok, 1), jnp.float32),
            pltpu.VMEM((N_DEV - 1, n_tok, 1), jnp.int32),
            pltpu.VMEM((N_DEV - 1, n_tok, h_dim), jnp.float32),
            pltpu.VMEM((N_DEV - 1, n_tok, h_dim), jnp.float32),
            pltpu.SemaphoreType.DMA((N_DEV - 1, 3)),
            pltpu.SemaphoreType.DMA((N_DEV - 1, 3)),
            pltpu.SemaphoreType.DMA((N_DEV - 1,)),
            pltpu.SemaphoreType.DMA((N_DEV - 1,)),
            pltpu.VMEM((n_tok, 1), jnp.float32),
        ],
        compiler_params=pltpu.CompilerParams(collective_id=0),
    )(x, router_W, route_idx, expert_W, shared_W)


# baseline (device time: 215747 ns/iter reference)
import jax
import jax.numpy as jnp
from jax import lax
from jax.experimental import pallas as pl
from jax.experimental.pallas import tpu as pltpu

N_DEV = 4
E_LOC = 4


def kernel(x, router_W, route_idx, expert_W, shared_W):
    n_tok, d_model = x.shape
    h_dim = shared_W.shape[1]

    def body(x_ref, rw_ref, ridx_ref, ew_ref, sw_ref, out_ref,
             xcomm, pcomm, rcomm, psend, precv,
             send_sems, recv_sems, psend_sems, precv_sems, pg_ref):
        my = lax.axis_index("i")
        left = lax.rem(my + N_DEV - 1, N_DEV)
        right = lax.rem(my + 1, N_DEV)

        bsem = pltpu.get_barrier_semaphore()
        for nbr in (left, right):
            pl.semaphore_signal(
                bsem, inc=1,
                device_id=(nbr,), device_id_type=pl.DeviceIdType.MESH,
            )
        pl.semaphore_wait(bsem, 2)

        xv = x_ref[...]
        scores = jnp.dot(xv, rw_ref[...], preferred_element_type=jnp.float32)
        smax = jnp.max(scores, axis=-1, keepdims=True)
        ex = jnp.exp(scores - smax)
        probs = ex / jnp.sum(ex, axis=-1, keepdims=True)
        iota = lax.broadcasted_iota(jnp.int32, scores.shape, 1)
        ridx = ridx_ref[...]
        pg_ref[...] = jnp.sum(
            jnp.where(iota == ridx, probs, 0.0), axis=-1, keepdims=True
        )

        def expert_partial(xo, po, ro):
            acc = None
            for j in range(E_LOC):
                gate = jnp.where(ro == E_LOC * my + j, po, 0.0)
                contrib = jnp.dot(
                    xo * gate, ew_ref[j], preferred_element_type=jnp.float32
                )
                acc = contrib if acc is None else acc + contrib
            return acc

        def make_hop(h):
            if h == 0:
                srcs = (x_ref, pg_ref, ridx_ref)
            else:
                srcs = (xcomm.at[h - 1], pcomm.at[h - 1], rcomm.at[h - 1])
            dsts = (xcomm.at[h], pcomm.at[h], rcomm.at[h])
            rdmas = []
            for k in range(3):
                rdmas.append(pltpu.make_async_remote_copy(
                    src_ref=srcs[k],
                    dst_ref=dsts[k],
                    send_sem=send_sems.at[h, k],
                    recv_sem=recv_sems.at[h, k],
                    device_id=(right,),
                    device_id_type=pl.DeviceIdType.MESH,
                ))
            return rdmas

        hop = make_hop(0)
        for r in hop:
            r.start()

        out_ref[...] = (
            jnp.dot(xv, sw_ref[...], preferred_element_type=jnp.float32)
            + expert_partial(xv, pg_ref[...], ridx)
        )

        partial_rdmas = []
        for h in range(N_DEV - 1):
            for r in hop:
                r.wait()
            if h < N_DEV - 2:
                hop = make_hop(h + 1)
                for r in hop:
                    r.start()
            origin = lax.rem(my + N_DEV - 1 - h, N_DEV)
            acc = expert_partial(xcomm[h], pcomm[h], rcomm[h])
            psend[h] = acc
            rp = pltpu.make_async_remote_copy(
                src_ref=psend.at[h],
                dst_ref=precv.at[h],
                send_sem=psend_sems.at[h],
                recv_sem=precv_sems.at[h],
                device_id=(origin,),
                device_id_type=pl.DeviceIdType.MESH,
            )
            rp.start()
            partial_rdmas.append(rp)

        for rp in partial_rdmas:
            rp.wait()
        out_ref[...] = out_ref[...] + precv[0] + precv[1] + precv[2]

    return pl.pallas_call(
        body,
        out_shape=jax.ShapeDtypeStruct((n_tok, h_dim), jnp.float32),
        in_specs=[pl.BlockSpec(memory_space=pltpu.VMEM)] * 5,
        out_specs=pl.BlockSpec(memory_space=pltpu.VMEM),
        scratch_shapes=[
            pltpu.VMEM((N_DEV - 1, n_tok, d_model), jnp.float32),
            pltpu.VMEM((N_DEV - 1, n_tok, 1), jnp.float32),
            pltpu.VMEM((N_DEV - 1, n_tok, 1), jnp.int32),
            pltpu.VMEM((N_DEV - 1, n_tok, h_dim), jnp.float32),
            pltpu.VMEM((N_DEV - 1, n_tok, h_dim), jnp.float32),
            pltpu.SemaphoreType.DMA((N_DEV - 1, 3)),
            pltpu.SemaphoreType.DMA((N_DEV - 1, 3)),
            pltpu.SemaphoreType.DMA((N_DEV - 1,)),
            pltpu.SemaphoreType.DMA((N_DEV - 1,)),
            pltpu.VMEM((n_tok, 1), jnp.float32),
        ],
        compiler_params=pltpu.CompilerParams(
            collective_id=0,
            vmem_limit_bytes=100 * 1024 * 1024,
        ),
    )(x, router_W, route_idx, expert_W, shared_W)


# device time: 137750 ns/iter; 1.5662x vs baseline; 1.5662x over previous
import jax
import jax.numpy as jnp
from jax import lax
from jax.experimental import pallas as pl
from jax.experimental.pallas import tpu as pltpu

N_DEV = 4
E_LOC = 4


def kernel(x, router_W, route_idx, expert_W, shared_W):
    n_tok, d_model = x.shape
    h_dim = shared_W.shape[1]

    def body(x_ref, rw_ref, ridx_ref, ew_ref, sw_ref, out_ref,
             xbf, ewbf, swbf,
             xcomm, pcomm, rcomm, psend, precv,
             send_sems, recv_sems, psend_sems, precv_sems, pg_ref):
        my = lax.axis_index("i")
        left = lax.rem(my + N_DEV - 1, N_DEV)
        right = lax.rem(my + 1, N_DEV)

        bsem = pltpu.get_barrier_semaphore()
        for nbr in (left, right):
            pl.semaphore_signal(
                bsem, inc=1,
                device_id=(nbr,), device_id_type=pl.DeviceIdType.MESH,
            )
        pl.semaphore_wait(bsem, 2)

        xv = x_ref[...]
        scores = jnp.dot(xv, rw_ref[...], preferred_element_type=jnp.float32)
        smax = jnp.max(scores, axis=-1, keepdims=True)
        ex = jnp.exp(scores - smax)
        probs = ex / jnp.sum(ex, axis=-1, keepdims=True)
        iota = lax.broadcasted_iota(jnp.int32, scores.shape, 1)
        ridx = ridx_ref[...]
        pg_ref[...] = jnp.sum(
            jnp.where(iota == ridx, probs, 0.0), axis=-1, keepdims=True
        )
        xbf[...] = xv.astype(jnp.bfloat16)
        ewbf[...] = ew_ref[...].astype(jnp.bfloat16)
        swbf[...] = sw_ref[...].astype(jnp.bfloat16)

        def expert_partial(xo, po, ro):
            pbf = po.astype(jnp.bfloat16)
            acc = None
            for j in range(E_LOC):
                gate = jnp.where(ro == E_LOC * my + j, pbf, jnp.bfloat16(0))
                contrib = jnp.dot(
                    xo * gate, ewbf[j], preferred_element_type=jnp.float32
                )
                acc = contrib if acc is None else acc + contrib
            return acc

        def make_hop(h):
            if h == 0:
                srcs = (xbf, pg_ref, ridx_ref)
            else:
                srcs = (xcomm.at[h - 1], pcomm.at[h - 1], rcomm.at[h - 1])
            dsts = (xcomm.at[h], pcomm.at[h], rcomm.at[h])
            rdmas = []
            for k in range(3):
                rdmas.append(pltpu.make_async_remote_copy(
                    src_ref=srcs[k],
                    dst_ref=dsts[k],
                    send_sem=send_sems.at[h, k],
                    recv_sem=recv_sems.at[h, k],
                    device_id=(right,),
                    device_id_type=pl.DeviceIdType.MESH,
                ))
            return rdmas

        hop = make_hop(0)
        for r in hop:
            r.start()

        out_ref[...] = (
            jnp.dot(xbf[...], swbf[...], preferred_element_type=jnp.float32)
            + expert_partial(xbf[...], pg_ref[...], ridx)
        )

        partial_rdmas = []
        for h in range(N_DEV - 1):
            for r in hop:
                r.wait()
            if h < N_DEV - 2:
                hop = make_hop(h + 1)
                for r in hop:
                    r.start()
            origin = lax.rem(my + N_DEV - 1 - h, N_DEV)
            acc = expert_partial(xcomm[h], pcomm[h], rcomm[h])
            psend[h] = acc.astype(jnp.bfloat16)
            rp = pltpu.make_async_remote_copy(
                src_ref=psend.at[h],
                dst_ref=precv.at[h],
                send_sem=psend_sems.at[h],
                recv_sem=precv_sems.at[h],
                device_id=(origin,),
                device_id_type=pl.DeviceIdType.MESH,
            )
            rp.start()
            partial_rdmas.append(rp)

        for rp in partial_rdmas:
            rp.wait()
        out_ref[...] = out_ref[...] + (
            precv[0].astype(jnp.float32)
            + precv[1].astype(jnp.float32)
            + precv[2].astype(jnp.float32)
        )

    return pl.pallas_call(
        body,
        out_shape=jax.ShapeDtypeStruct((n_tok, h_dim), jnp.float32),
        in_specs=[pl.BlockSpec(memory_space=pltpu.VMEM)] * 5,
        out_specs=pl.BlockSpec(memory_space=pltpu.VMEM),
        scratch_shapes=[
            pltpu.VMEM((n_tok, d_model), jnp.bfloat16),
            pltpu.VMEM((E_LOC, d_model, h_dim), jnp.bfloat16),
            pltpu.VMEM((d_model, h_dim), jnp.bfloat16),
            pltpu.VMEM((N_DEV - 1, n_tok, d_model), jnp.bfloat16),
            pltpu.VMEM((N_DEV - 1, n_tok, 1), jnp.float32),
            pltpu.VMEM((N_DEV - 1, n_tok, 1), jnp.int32),
            pltpu.VMEM((N_DEV - 1, n_tok, h_dim), jnp.bfloat16),
            pltpu.VMEM((N_DEV - 1, n_tok, h_dim), jnp.bfloat16),
            pltpu.SemaphoreType.DMA((N_DEV - 1, 3)),
            pltpu.SemaphoreType.DMA((N_DEV - 1, 3)),
            pltpu.SemaphoreType.DMA((N_DEV - 1,)),
            pltpu.SemaphoreType.DMA((N_DEV - 1,)),
            pltpu.VMEM((n_tok, 1), jnp.float32),
        ],
        compiler_params=pltpu.CompilerParams(
            collective_id=0,
            vmem_limit_bytes=100 * 1024 * 1024,
        ),
    )(x, router_W, route_idx, expert_W, shared_W)
